# baseline (speedup 1.0000x reference)
import functools

import jax
import jax.numpy as jnp
from jax import lax
from jax.experimental import pallas as pl
from jax.experimental.pallas import tpu as pltpu
from jax.experimental.pallas import tpu_sc as plsc

VOCAB = 1000000
DIM = 32
SEQ = 50
BATCH = 4096
NC = 2
NS = 16
LANES = 16
CPT = 2 * SEQ
MAIN = 999424
PADW = 1000448
KCH = 4096
NCHK = MAIN // KCH
HALF = 500000


def _det_body(tT_hbm, tail_hbm, outf_hbm, stg_v, ssem0, ssem1, wsem):
    cid = lax.axis_index("c")
    sid = lax.axis_index("s")
    ssems = (ssem0, ssem1)
    nmine = (NCHK - 1 - sid) // NS + 1

    def my_t(m):
        return m * NS + sid

    def run_group(g):
        def stage(m, b):
            c0 = my_t(m) * KCH
            return pltpu.make_async_copy(
                tT_hbm.at[pl.ds(8 * g, 8), pl.ds(c0, KCH)],
                stg_v.at[b, pl.ds(0, 8), pl.ds(0, KCH)],
                ssems[b],
            )

        def wr(m, b, k):
            c0 = my_t(m) * KCH
            dl = k // (KCH // 1024)
            j = k % (KCH // 1024)
            return pltpu.make_async_copy(
                stg_v.at[b, dl, pl.ds(j * 1024, 1024)],
                outf_hbm.at[pl.ds((8 * g + dl) * PADW + c0 + j * 1024, 1024)],
                wsem,
            )

        nw = 8 * (KCH // 1024)

        stage(0, 0).start()

        def task_pair(mp, c):
            for b in range(2):
                m = mp * 2 + b

                @pl.when(m < nmine)
                def _(m=m, b=b):
                    stage(m, b).wait()

                    @pl.when(m + 1 < nmine)
                    def _():
                        stage(m + 1, 1 - b).start()

                    def wfire(k, c2):
                        wr(m, b, k).start()
                        return c2

                    def wdrain(k, c2):
                        wr(m, b, k).wait()
                        return c2

                    lax.fori_loop(0, nw, wfire, 0)
                    lax.fori_loop(0, nw, wdrain, 0)

            return c

        lax.fori_loop(0, (nmine + 1) // 2, task_pair, 0)

        @pl.when(sid == NS - 1)
        def _():
            def tstage():
                return pltpu.make_async_copy(
                    tail_hbm.at[pl.ds(8 * g, 8), pl.ds(0, 1024)],
                    stg_v.at[0, pl.ds(0, 8), pl.ds(0, 1024)],
                    ssem0,
                )

            tstage().start()
            tstage().wait()

            def twr(k):
                return pltpu.make_async_copy(
                    stg_v.at[0, k, pl.ds(0, 1024)],
                    outf_hbm.at[pl.ds((8 * g + k) * PADW + MAIN, 1024)],
                    wsem,
                )

            def twfire(k, c):
                twr(k).start()
                return c

            def twdrain(k, c):
                twr(k).wait()
                return c

            lax.fori_loop(0, 8, twfire, 0)
            lax.fori_loop(0, 8, twdrain, 0)

    for gg in range(2):
        run_group(cid * 2 + gg)


@jax.jit
def _detile(tT, tail):
    mesh = plsc.VectorSubcoreMesh(core_axis_name="c", subcore_axis_name="s")
    k = functools.partial(
        pl.kernel,
        mesh=mesh,
        out_type=jax.ShapeDtypeStruct((DIM * PADW,), jnp.float32),
        compiler_params=pltpu.CompilerParams(
            needs_layout_passes=False, use_tc_tiling_on_sc=True
        ),
        scratch_types=[
            pltpu.VMEM((2, 8, KCH), jnp.float32),
            pltpu.SemaphoreType.DMA,
            pltpu.SemaphoreType.DMA,
            pltpu.SemaphoreType.DMA,
        ],
    )(_det_body)
    return k(tT, tail)


def _plane_body(qT_hbm, tpad_hbm, out_hbm, qblk_v, vals_v, bid_v, gst_v,
                zv_v, shlo, shhi, qsem, slo_sem, shi_sem, gsem, osem):
    cid = lax.axis_index("c")
    sid = lax.axis_index("s")
    lane_iota = lax.iota(jnp.int32, LANES)
    NB = CPT * 128

    def chunk_sb(i):
        return i // 2, sid * 256 + (i % 2) * 128

    def q_copy(i):
        s, b = chunk_sb(i)
        return pltpu.make_async_copy(
            qT_hbm.at[s, pl.ds(b, 128)], qblk_v.at[pl.ds(i * 128, 128)], qsem
        )

    def qfire(i, c):
        q_copy(i).start()
        return c

    def qdrain(i, c):
        q_copy(i).wait()
        return c

    lax.fori_loop(0, CPT, qfire, 0)
    lax.fori_loop(0, CPT, qdrain, 0)

    zv_v[pl.ds(0, LANES)] = jnp.zeros((LANES,), jnp.float32)
    zeros_i = jnp.zeros((LANES,), jnp.int32)

    def prefill(k, c):
        bid_v[pl.ds(k * LANES, LANES)] = zeros_i
        return c

    lax.fori_loop(0, (NB + 256) // LANES, prefill, 0)

    def count_lo(g, c):
        v = qblk_v[pl.ds(g * LANES, LANES)]
        nlo = plsc.all_reduce_population_count(v < HALF)[0]
        return c + nlo

    clo = lax.fori_loop(0, NB // LANES, count_lo, 0)
    chi = NB - clo
    hibase = (clo + 127) // 128 * 128

    def binify(g, carry):
        olo, ohi = carry
        v = qblk_v[pl.ds(g * LANES, LANES)]
        mlo = v < HALF
        cell_lo = jnp.where(v == 0, 0, v + 8)
        cell_hi = v - (HALF - 8)
        plsc.store_compressed(bid_v.at[pl.ds(olo, LANES)], cell_lo, mask=mlo)
        plsc.store_compressed(bid_v.at[pl.ds(ohi, LANES)], cell_hi, mask=~mlo)
        nlo = plsc.all_reduce_population_count(mlo)[0]
        return olo + nlo, ohi + (LANES - nlo)

    lax.fori_loop(0, NB // LANES, binify, (0, hibase))
    nch_lo = (clo + 127) // 128
    nch_hi = (chi + 127) // 128

    @pl.when(sid == 0)
    def _():
        pltpu.sync_copy(zv_v.at[pl.ds(0, 8)], shlo.at[pl.ds(0, 8)])
        pltpu.sync_copy(zv_v.at[pl.ds(0, 8)], shhi.at[pl.ds(0, 8)])

    def stage(p, e):
        sh = (shlo, shhi)[e]
        sem = (slo_sem, shi_sem)[e]
        return pltpu.make_async_copy(
            tpad_hbm.at[cid * NS + p, pl.ds(e * HALF, HALF)],
            sh.at[pl.ds(8, HALF)],
            sem,
        )

    @pl.when(sid == 0)
    def _():
        stage(0, 0).start()
        stage(0, 1).start()

    def gather(c, e, base):
        sh = (shlo, shhi)[e]
        return pltpu.make_async_copy(
            sh.at[bid_v.at[pl.ds(base + c * 128, 128)]],
            gst_v.at[pl.ds(base + c * 128, 128)],
            gsem,
        )

    def plane_body(p, carry):
        d = cid * NS + p

        for e in range(2):
            nch = (nch_lo, nch_hi)[e]
            base = (0, hibase)[e]

            @pl.when(sid == 0)
            def _(e=e):
                stage(p, e).wait()

            plsc.subcore_barrier()

            def gfire(c, c2, e=e, base=base):
                gather(c, e, base).start()
                return c2

            def gdrain(c, c2, e=e, base=base):
                gather(c, e, base).wait()
                return c2

            lax.fori_loop(0, nch, gfire, 0)
            lax.fori_loop(0, nch, gdrain, 0)
            plsc.subcore_barrier()

            @pl.when((sid == 0) & (p + 1 < NS))
            def _(e=e):
                stage(p + 1, e).start()

        def merge(g, carry2):
            c2lo, c2hi = carry2
            v = qblk_v[pl.ds(g * LANES, LANES)]
            mlo = v < HALF
            xlo = plsc.load_expanded(gst_v.at[pl.ds(c2lo, LANES)], mask=mlo)
            xhi = plsc.load_expanded(gst_v.at[pl.ds(c2hi, LANES)], mask=~mlo)
            vals_v[pl.ds(g * LANES, LANES)] = jnp.where(mlo, xlo, xhi)
            nlo = plsc.all_reduce_population_count(mlo)[0]
            return c2lo + nlo, c2hi + (LANES - nlo)

        lax.fori_loop(0, NB // LANES, merge, (0, hibase))

        def store(i):
            s, b = chunk_sb(i)
            return pltpu.make_async_copy(
                vals_v.at[pl.ds(i * 128, 128)],
                out_hbm.at[s, d // 8, b // 128, d % 8],
                osem,
            )

        def sdrain(i, c2):
            store(i).wait()
            return c2

        for i in range(CPT):
            store(i).start()
        lax.fori_loop(0, CPT, sdrain, 0)
        return carry

    lax.fori_loop(0, NS, plane_body, 0)


@jax.jit
def _plane_gather(tpad, qT):
    mesh = plsc.VectorSubcoreMesh(core_axis_name="c", subcore_axis_name="s")
    k = functools.partial(
        pl.kernel,
        mesh=mesh,
        out_type=jax.ShapeDtypeStruct((SEQ, DIM // 8, BATCH // 128, 8, 128), jnp.float32),
        compiler_params=pltpu.CompilerParams(
            needs_layout_passes=False, use_tc_tiling_on_sc=False
        ),
        scratch_types=[
            pltpu.VMEM((CPT * 128,), jnp.int32),
            pltpu.VMEM((CPT * 128,), jnp.float32),
            pltpu.VMEM((CPT * 128 + 256,), jnp.int32),
            pltpu.VMEM((CPT * 128 + 256,), jnp.float32),
            pltpu.VMEM((LANES,), jnp.float32),
            pltpu.VMEM_SHARED((HALF + 8,), jnp.float32),
            pltpu.VMEM_SHARED((HALF + 8,), jnp.float32),
            pltpu.SemaphoreType.DMA,
            pltpu.SemaphoreType.DMA,
            pltpu.SemaphoreType.DMA,
            pltpu.SemaphoreType.DMA,
            pltpu.SemaphoreType.DMA,
        ],
    )(_plane_body)
    return k(qT, tpad)


def kernel(q, q_len, table):
    tT = table.T
    tail = jnp.pad(tT[:, MAIN:], ((0, 0), (0, PADW - VOCAB)))
    tpad = _detile(tT, tail).reshape(DIM, PADW)
    out5 = _plane_gather(tpad, q.T)
    return out5.transpose(2, 4, 0, 1, 3).reshape(BATCH, SEQ, DIM)


# speedup vs baseline: 1.1514x; 1.1514x over previous
import functools

import jax
import jax.numpy as jnp
from jax import lax
from jax.experimental import pallas as pl
from jax.experimental.pallas import tpu as pltpu
from jax.experimental.pallas import tpu_sc as plsc

VOCAB = 1000000
DIM = 32
SEQ = 50
BATCH = 4096
NC = 2
NS = 16
LANES = 16
CPT = 2 * SEQ
MAIN = 999424
PADW = 1000448
KCH = 4096
NCHK = MAIN // KCH


def _det_body(tT_hbm, tail_hbm, outf_hbm, stg_v, ssem0, ssem1, wsem):
    cid = lax.axis_index("c")
    sid = lax.axis_index("s")
    ssems = (ssem0, ssem1)
    nmine = (NCHK - 1 - sid) // NS + 1

    def my_t(m):
        return m * NS + sid

    def run_group(g):
        def stage(m, b):
            c0 = my_t(m) * KCH
            return pltpu.make_async_copy(
                tT_hbm.at[pl.ds(8 * g, 8), pl.ds(c0, KCH)],
                stg_v.at[b, pl.ds(0, 8), pl.ds(0, KCH)],
                ssems[b],
            )

        def wr(m, b, k):
            c0 = my_t(m) * KCH
            dl = k // (KCH // 1024)
            j = k % (KCH // 1024)
            return pltpu.make_async_copy(
                stg_v.at[b, dl, pl.ds(j * 1024, 1024)],
                outf_hbm.at[pl.ds((8 * g + dl) * PADW + c0 + j * 1024, 1024)],
                wsem,
            )

        nw = 8 * (KCH // 1024)

        stage(0, 0).start()

        def task_pair(mp, c):
            for b in range(2):
                m = mp * 2 + b

                @pl.when(m < nmine)
                def _(m=m, b=b):
                    stage(m, b).wait()

                    @pl.when(m + 1 < nmine)
                    def _():
                        stage(m + 1, 1 - b).start()

                    def wfire(k, c2):
                        wr(m, b, k).start()
                        return c2

                    def wdrain(k, c2):
                        wr(m, b, k).wait()
                        return c2

                    lax.fori_loop(0, nw, wfire, 0)
                    lax.fori_loop(0, nw, wdrain, 0)

            return c

        lax.fori_loop(0, (nmine + 1) // 2, task_pair, 0)

        @pl.when(sid == NS - 1)
        def _():
            def tstage():
                return pltpu.make_async_copy(
                    tail_hbm.at[pl.ds(8 * g, 8), pl.ds(0, 1024)],
                    stg_v.at[0, pl.ds(0, 8), pl.ds(0, 1024)],
                    ssem0,
                )

            tstage().start()
            tstage().wait()

            def twr(k):
                return pltpu.make_async_copy(
                    stg_v.at[0, k, pl.ds(0, 1024)],
                    outf_hbm.at[pl.ds((8 * g + k) * PADW + MAIN, 1024)],
                    wsem,
                )

            def twfire(k, c):
                twr(k).start()
                return c

            def twdrain(k, c):
                twr(k).wait()
                return c

            lax.fori_loop(0, 8, twfire, 0)
            lax.fori_loop(0, 8, twdrain, 0)

    for gg in range(2):
        run_group(cid * 2 + gg)


@jax.jit
def _detile(tT, tail):
    mesh = plsc.VectorSubcoreMesh(core_axis_name="c", subcore_axis_name="s")
    k = functools.partial(
        pl.kernel,
        mesh=mesh,
        out_type=jax.ShapeDtypeStruct((DIM * PADW,), jnp.float32),
        compiler_params=pltpu.CompilerParams(
            needs_layout_passes=False, use_tc_tiling_on_sc=True
        ),
        scratch_types=[
            pltpu.VMEM((2, 8, KCH), jnp.float32),
            pltpu.SemaphoreType.DMA,
            pltpu.SemaphoreType.DMA,
            pltpu.SemaphoreType.DMA,
        ],
    )(_det_body)
    return k(tT, tail)


def _plane_body(qT_hbm, tpad_hbm, out_hbm, qblk_v, vals_v, zv_v, shared, qsem, gsem, osem):
    cid = lax.axis_index("c")
    sid = lax.axis_index("s")

    def chunk_sb(i):
        return i // 2, sid * 256 + (i % 2) * 128

    def q_copy(i):
        s, b = chunk_sb(i)
        return pltpu.make_async_copy(
            qT_hbm.at[s, pl.ds(b, 128)], qblk_v.at[pl.ds(i * 128, 128)], qsem
        )

    def qfire(i, c):
        q_copy(i).start()
        return c

    def qdrain(i, c):
        q_copy(i).wait()
        return c

    lax.fori_loop(0, CPT, qfire, 0)
    lax.fori_loop(0, CPT, qdrain, 0)

    zv_v[pl.ds(0, LANES)] = jnp.zeros((LANES,), jnp.float32)

    def remap(g, c):
        v = qblk_v[pl.ds(g * LANES, LANES)]
        qblk_v[pl.ds(g * LANES, LANES)] = jnp.where(v == 0, 0, v + 8)
        return c

    lax.fori_loop(0, CPT * 128 // LANES, remap, 0)

    @pl.when(sid == 0)
    def _():
        pltpu.sync_copy(zv_v.at[pl.ds(0, 8)], shared.at[pl.ds(0, 8)])

    def gather(i):
        return pltpu.make_async_copy(
            shared.at[qblk_v.at[pl.ds(i * 128, 128)]],
            vals_v.at[pl.ds(i * 128, 128)],
            gsem,
        )

    def stage(p):
        pltpu.sync_copy(
            tpad_hbm.at[cid * NS + p, pl.ds(0, VOCAB)], shared.at[pl.ds(8, VOCAB)]
        )

    @pl.when(sid == 0)
    def _():
        stage(0)

    def plane_body(p, carry):
        d = cid * NS + p
        plsc.subcore_barrier()

        def store(i):
            s, b = chunk_sb(i)
            return pltpu.make_async_copy(
                vals_v.at[pl.ds(i * 128, 128)],
                out_hbm.at[s, d // 8, b // 128, d % 8],
                osem,
            )

        def gfire(i, c):
            gather(i).start()
            return c

        def gdrain(i, c):
            gather(i).wait()
            return c

        def sfire(i, c):
            store(i).start()
            return c

        def sdrain(i, c):
            store(i).wait()
            return c

        for i in range(CPT):
            gather(i).start()
        lax.fori_loop(0, CPT, gdrain, 0)
        plsc.subcore_barrier()

        @pl.when((sid == 0) & (p + 1 < NS))
        def _():
            stage(p + 1)

        for i in range(CPT):
            store(i).start()
        lax.fori_loop(0, CPT, sdrain, 0)
        return carry

    lax.fori_loop(0, NS, plane_body, 0)


@jax.jit
def _plane_gather(tpad, qT):
    mesh = plsc.VectorSubcoreMesh(core_axis_name="c", subcore_axis_name="s")
    k = functools.partial(
        pl.kernel,
        mesh=mesh,
        out_type=jax.ShapeDtypeStruct((SEQ, DIM // 8, BATCH // 128, 8, 128), jnp.float32),
        compiler_params=pltpu.CompilerParams(
            needs_layout_passes=False, use_tc_tiling_on_sc=False
        ),
        scratch_types=[
            pltpu.VMEM((CPT * 128,), jnp.int32),
            pltpu.VMEM((CPT * 128,), jnp.float32),
            pltpu.VMEM((LANES,), jnp.float32),
            pltpu.VMEM_SHARED((VOCAB + 8,), jnp.float32),
            pltpu.SemaphoreType.DMA,
            pltpu.SemaphoreType.DMA,
            pltpu.SemaphoreType.DMA,
        ],
    )(_plane_body)
    return k(qT, tpad)


def kernel(q, q_len, table):
    tT = table.T
    tail = jnp.pad(tT[:, MAIN:], ((0, 0), (0, PADW - VOCAB)))
    tpad = _detile(tT, tail).reshape(DIM, PADW)
    out5 = _plane_gather(tpad, q.T)
    return out5.transpose(2, 4, 0, 1, 3).reshape(BATCH, SEQ, DIM)
